# baseline (device time: 73081 ns/iter reference)
import functools

import jax
import jax.numpy as jnp
from jax import lax
from jax.experimental import pallas as pl
from jax.experimental.pallas import tpu as pltpu

N_DEV = 16
B_PER = 2
SQ = 256
SKV = 256
HQ = 64
H_PER = 4
DH = 64
D_MODEL = 512
HD_PER = H_PER * DH
WINDOW = 128
N_RIGHT = N_DEV // 2 - 1
N_LEFT = N_DEV // 2

QSCALE = 10.0 / 16384.0


def _mod(v):
    return lax.rem(v + 2 * N_DEV, N_DEV)


def kernel(x, Wq, K_ext, V_ext, Wo):
    my = lax.axis_index("i")

    x_b = x.astype(jnp.bfloat16)
    w_pack = jnp.concatenate([Wq, Wo.T], axis=0)
    w_pack = jnp.clip(jnp.rint(w_pack / QSCALE), -127, 127).astype(jnp.int8)
    k_loc = lax.dynamic_slice_in_dim(K_ext, my * B_PER, B_PER, axis=0)
    v_loc = lax.dynamic_slice_in_dim(V_ext, my * B_PER, B_PER, axis=0)
    k_in = k_loc.astype(jnp.bfloat16).reshape(B_PER, SKV, HQ * DH)
    v_in = v_loc.astype(jnp.bfloat16).reshape(B_PER, SKV, HQ * DH)

    def body(x_ref, w_ref, k_ref, v_ref, out_ref,
             w_all, ssem_r, rsem_r, ssem_l, rsem_l):
        my_i = lax.axis_index("i")
        right = lax.rem(my_i + 1, N_DEV)
        left = lax.rem(my_i + N_DEV - 1, N_DEV)

        barrier = pltpu.get_barrier_semaphore()
        for nbr in (left, right):
            pl.semaphore_signal(barrier, inc=1, device_id=(nbr,),
                                device_id_type=pl.DeviceIdType.MESH)
        pl.semaphore_wait(barrier, 2)

        w_all[pl.ds(my_i, 1)] = w_ref[...][None]
        out_ref[...] = jnp.zeros_like(out_ref)

        qi = lax.broadcasted_iota(jnp.int32, (SQ, SKV), 0)
        ki = lax.broadcasted_iota(jnp.int32, (SQ, SKV), 1)
        mask = jnp.abs(qi - ki) <= WINDOW

        def accumulate(o):
            wq_c = w_all[pl.ds(o, 1), pl.ds(0, D_MODEL)][0]
            wq_c = wq_c.astype(jnp.bfloat16)
            woT_c = w_all[pl.ds(o, 1), pl.ds(D_MODEL, D_MODEL)][0]
            woT_c = woT_c.astype(jnp.bfloat16)
            for b in range(B_PER):
                xb = x_ref[b]
                q = jnp.dot(xb, wq_c,
                            preferred_element_type=jnp.float32)
                q = (q * (QSCALE * 0.125)).astype(jnp.bfloat16)
                ctxs = []
                for j in range(H_PER // 2):
                    off = pl.multiple_of((o * 2 + j) * (2 * DH), 2 * DH)
                    kp = k_ref[b, :, pl.ds(off, 2 * DH)]
                    vp = v_ref[b, :, pl.ds(off, 2 * DH)]
                    for t in range(2):
                        hh = 2 * j + t
                        kh = kp[:, t * DH:(t + 1) * DH]
                        vh = vp[:, t * DH:(t + 1) * DH]
                        qh = q[:, hh * DH:(hh + 1) * DH]
                        s = lax.dot_general(
                            qh, kh, (((1,), (1,)), ((), ())),
                            preferred_element_type=jnp.float32)
                        w = jnp.where(mask, jnp.exp(s), 0.0)
                        recip = 1.0 / jnp.sum(w, axis=1, keepdims=True)
                        ctx_h = jnp.dot(w.astype(jnp.bfloat16), vh,
                                        preferred_element_type=jnp.float32)
                        ctxs.append((ctx_h * recip).astype(jnp.bfloat16))
                ctx = jnp.concatenate(ctxs, axis=1)
                contrib = lax.dot_general(
                    ctx, woT_c, (((1,), (1,)), ((), ())),
                    preferred_element_type=jnp.float32)
                out_ref[b] = out_ref[b] + contrib

        def rdma(o, part, dest, ssem, rsem):
            sl = pl.ds(part * D_MODEL, D_MODEL)
            return pltpu.make_async_remote_copy(
                src_ref=w_all.at[o, sl], dst_ref=w_all.at[o, sl],
                send_sem=ssem.at[o, part], recv_sem=rsem.at[o, part],
                device_id=(dest,), device_id_type=pl.DeviceIdType.MESH)

        def start_both(o, dest, ssem, rsem):
            a = rdma(o, 0, dest, ssem, rsem)
            b = rdma(o, 1, dest, ssem, rsem)
            a.start()
            b.start()
            return a, b

        def wait_recv_both(o, dest, ssem, rsem):
            rdma(o, 0, dest, ssem, rsem).wait_recv()
            rdma(o, 1, dest, ssem, rsem).wait_recv()

        sr_a, sr_b = start_both(my_i, right, ssem_r, rsem_r)
        sl_a, sl_b = start_both(my_i, left, ssem_l, rsem_l)
        accumulate(my_i)
        wait_recv_both(_mod(my_i - 1), right, ssem_r, rsem_r)
        wait_recv_both(_mod(my_i + 1), left, ssem_l, rsem_l)
        for d in (sr_a, sr_b, sl_a, sl_b):
            d.wait_send()

        def hop(h, carry):
            o_r = _mod(my_i - h)
            o_l = _mod(my_i + h)

            @pl.when(h < N_RIGHT)
            def _():
                start_both(o_r, right, ssem_r, rsem_r)

            sl_ah, sl_bh = start_both(o_l, left, ssem_l, rsem_l)

            accumulate(o_r)
            accumulate(o_l)

            @pl.when(h < N_RIGHT)
            def _():
                wait_recv_both(_mod(my_i - h - 1), right, ssem_r, rsem_r)
                rdma(o_r, 0, right, ssem_r, rsem_r).wait_send()
                rdma(o_r, 1, right, ssem_r, rsem_r).wait_send()

            wait_recv_both(_mod(my_i + h + 1), left, ssem_l, rsem_l)
            sl_ah.wait_send()
            sl_bh.wait_send()
            return carry

        lax.fori_loop(1, N_LEFT, hop, 0)
        accumulate(_mod(my_i + N_LEFT))
        out_ref[...] = out_ref[...] * QSCALE

        @functools.partial(pl.run_scoped,
                           exit_sem=pltpu.SemaphoreType.REGULAR)
        def _(exit_sem):
            for nbr in (left, right):
                pl.semaphore_signal(exit_sem, inc=1, device_id=(nbr,),
                                    device_id_type=pl.DeviceIdType.MESH)
            pl.semaphore_wait(exit_sem, 2)

    return pl.pallas_call(
        body,
        out_shape=jax.ShapeDtypeStruct((B_PER, SQ, D_MODEL), jnp.float32),
        in_specs=[pl.BlockSpec(memory_space=pltpu.VMEM)] * 4,
        out_specs=pl.BlockSpec(memory_space=pltpu.VMEM),
        scratch_shapes=[
            pltpu.VMEM((N_DEV, 2 * D_MODEL, HD_PER), jnp.int8),
            pltpu.SemaphoreType.DMA((N_DEV, 2)),
            pltpu.SemaphoreType.DMA((N_DEV, 2)),
            pltpu.SemaphoreType.DMA((N_DEV, 2)),
            pltpu.SemaphoreType.DMA((N_DEV, 2)),
        ],
        compiler_params=pltpu.CompilerParams(collective_id=0),
    )(x_b, w_pack, k_in, v_in)


# device time: 59818 ns/iter; 1.2217x vs baseline; 1.2217x over previous
import functools

import jax
import jax.numpy as jnp
from jax import lax
from jax.experimental import pallas as pl
from jax.experimental.pallas import tpu as pltpu

N_DEV = 16
B_PER = 2
SQ = 256
SKV = 256
HQ = 64
H_PER = 4
DH = 64
D_MODEL = 512
HD_PER = H_PER * DH
WINDOW = 128
N_RIGHT = N_DEV // 2 - 1
N_LEFT = N_DEV // 2

QSCALE = 10.0 / 16384.0

RING = (0, 4, 8, 12, 15, 11, 7, 3, 2, 6, 10, 14, 13, 9, 5, 1)


def _mod(v):
    return lax.rem(v + 2 * N_DEV, N_DEV)


def kernel(x, Wq, K_ext, V_ext, Wo):
    my = lax.axis_index("i")

    x_b = x.astype(jnp.bfloat16)
    w_pack = jnp.concatenate([Wq, Wo.T], axis=0)
    w_pack = jnp.clip(jnp.rint(w_pack / QSCALE), -127, 127).astype(jnp.int8)
    k_loc = lax.dynamic_slice_in_dim(K_ext, my * B_PER, B_PER, axis=0)
    v_loc = lax.dynamic_slice_in_dim(V_ext, my * B_PER, B_PER, axis=0)
    k_t = k_loc.transpose(0, 2, 3, 1).reshape(B_PER * HQ, DH, SKV)
    k_t = k_t.astype(jnp.bfloat16)
    v_t = v_loc.transpose(0, 2, 1, 3).reshape(B_PER * HQ, SKV, DH)
    v_t = v_t.astype(jnp.bfloat16)

    def body(x_ref, w_ref, k_ref, v_ref, out_ref,
             w_all, ssem_r, rsem_r, ssem_l, rsem_l):
        my_i = lax.axis_index("i")
        pos_iota = lax.broadcasted_iota(jnp.int32, (1, N_DEV), 1)
        seg = pos_iota // 4
        rr = lax.rem(pos_iota, 4)
        zz = jnp.where(lax.rem(seg, 2) == 0, rr, 3 - rr)
        ring_arr = 4 * zz + lax.rem(4 - seg, 4)
        my_pos = jnp.sum(jnp.where(ring_arr == my_i, pos_iota, 0))

        def ring_at(d):
            return jnp.sum(jnp.where(pos_iota == _mod(d), ring_arr, 0))

        right = ring_at(my_pos + 1)
        left = ring_at(my_pos - 1)

        barrier = pltpu.get_barrier_semaphore()
        for nbr in (left, right):
            pl.semaphore_signal(barrier, inc=1, device_id=(nbr,),
                                device_id_type=pl.DeviceIdType.MESH)
        pl.semaphore_wait(barrier, 2)

        w_all[pl.ds(my_i, 1)] = w_ref[...][None]
        out_ref[...] = jnp.zeros_like(out_ref)

        qi = lax.broadcasted_iota(jnp.int32, (SQ, SKV), 0)
        ki = lax.broadcasted_iota(jnp.int32, (SQ, SKV), 1)
        mask = jnp.abs(qi - ki) <= WINDOW

        def accumulate(o):
            wq_c = w_all[pl.ds(o, 1), pl.ds(0, D_MODEL)][0]
            wq_c = wq_c.astype(jnp.bfloat16)
            woT_c = w_all[pl.ds(o, 1), pl.ds(D_MODEL, D_MODEL)][0]
            woT_c = woT_c.astype(jnp.bfloat16)
            for b in range(B_PER):
                xb = x_ref[b]
                q = jnp.dot(xb, wq_c,
                            preferred_element_type=jnp.float32)
                q = (q * (QSCALE * 0.125)).astype(jnp.bfloat16)
                ctxs = []
                for hh in range(H_PER):
                    idx = b * HQ + o * H_PER + hh
                    kh = k_ref[pl.ds(idx, 1)][0]
                    vh = v_ref[pl.ds(idx, 1)][0]
                    qh = q[:, hh * DH:(hh + 1) * DH]
                    s = jnp.dot(qh, kh, preferred_element_type=jnp.float32)
                    w = jnp.where(mask, jnp.exp(s), 0.0)
                    recip = 1.0 / jnp.sum(w, axis=1, keepdims=True)
                    ctx_h = jnp.dot(w.astype(jnp.bfloat16), vh,
                                    preferred_element_type=jnp.float32)
                    ctxs.append((ctx_h * recip).astype(jnp.bfloat16))
                ctx = jnp.concatenate(ctxs, axis=1)
                contrib = lax.dot_general(
                    ctx, woT_c, (((1,), (1,)), ((), ())),
                    preferred_element_type=jnp.float32)
                out_ref[b] = out_ref[b] + contrib * QSCALE

        def rdma(o, part, dest, ssem, rsem):
            sl = pl.ds(part * D_MODEL, D_MODEL)
            return pltpu.make_async_remote_copy(
                src_ref=w_all.at[o, sl], dst_ref=w_all.at[o, sl],
                send_sem=ssem.at[o, part], recv_sem=rsem.at[o, part],
                device_id=(dest,), device_id_type=pl.DeviceIdType.MESH)

        def start_both(o, dest, ssem, rsem):
            a = rdma(o, 0, dest, ssem, rsem)
            b = rdma(o, 1, dest, ssem, rsem)
            a.start()
            b.start()
            return a, b

        def wait_recv_both(o, dest, ssem, rsem):
            rdma(o, 0, dest, ssem, rsem).wait_recv()
            rdma(o, 1, dest, ssem, rsem).wait_recv()

        sr_a, sr_b = start_both(my_i, right, ssem_r, rsem_r)
        sl_a, sl_b = start_both(my_i, left, ssem_l, rsem_l)
        accumulate(my_i)
        wait_recv_both(left, right, ssem_r, rsem_r)
        wait_recv_both(right, left, ssem_l, rsem_l)
        for d in (sr_a, sr_b, sl_a, sl_b):
            d.wait_send()

        def hop(h, carry):
            o_r = ring_at(my_pos - h)
            o_l = ring_at(my_pos + h)

            @pl.when(h < N_RIGHT)
            def _():
                start_both(o_r, right, ssem_r, rsem_r)

            sl_ah, sl_bh = start_both(o_l, left, ssem_l, rsem_l)

            accumulate(o_r)
            accumulate(o_l)

            @pl.when(h < N_RIGHT)
            def _():
                wait_recv_both(ring_at(my_pos - h - 1), right,
                               ssem_r, rsem_r)
                rdma(o_r, 0, right, ssem_r, rsem_r).wait_send()
                rdma(o_r, 1, right, ssem_r, rsem_r).wait_send()

            wait_recv_both(ring_at(my_pos + h + 1), left, ssem_l, rsem_l)
            sl_ah.wait_send()
            sl_bh.wait_send()
            return carry

        lax.fori_loop(1, N_LEFT, hop, 0)
        accumulate(ring_at(my_pos + N_LEFT))

        @functools.partial(pl.run_scoped,
                           exit_sem=pltpu.SemaphoreType.REGULAR)
        def _(exit_sem):
            for nbr in (left, right):
                pl.semaphore_signal(exit_sem, inc=1, device_id=(nbr,),
                                    device_id_type=pl.DeviceIdType.MESH)
            pl.semaphore_wait(exit_sem, 2)

    return pl.pallas_call(
        body,
        out_shape=jax.ShapeDtypeStruct((B_PER, SQ, D_MODEL), jnp.float32),
        in_specs=[pl.BlockSpec(memory_space=pltpu.VMEM)] * 4,
        out_specs=pl.BlockSpec(memory_space=pltpu.VMEM),
        scratch_shapes=[
            pltpu.VMEM((N_DEV, 2 * D_MODEL, HD_PER), jnp.int8),
            pltpu.SemaphoreType.DMA((N_DEV, 2)),
            pltpu.SemaphoreType.DMA((N_DEV, 2)),
            pltpu.SemaphoreType.DMA((N_DEV, 2)),
            pltpu.SemaphoreType.DMA((N_DEV, 2)),
        ],
        compiler_params=pltpu.CompilerParams(collective_id=0),
    )(x_b, w_pack, k_t, v_t)
